# baseline (speedup 1.0000x reference)
import jax
import jax.numpy as jnp
from jax.experimental import pallas as pl
from jax.experimental.pallas import tpu as pltpu


def _pool2x2(r):
    H, W, B = r.shape
    rr = r.reshape(H // 2, 2, W, B)
    a = jnp.maximum(rr[:, 0], rr[:, 1])
    aa = a.reshape(H // 2, W // 2, 2, B)
    return jnp.maximum(aa[:, :, 0, :], aa[:, :, 1, :])


def _fernet_kernel(x_ref, w1_ref, b1_ref, w2_ref, b2_ref, w3_ref, b3_ref,
                   f1w_ref, f1b_ref, f2w_ref, f2b_ref, f3w_ref, f3b_ref,
                   o_ref, a1_ref, a2_ref, a3_ref):
    B = x_ref.shape[-1]
    xb = x_ref[...]

    rows = [xb[kh:kh + 44, :, :] for kh in range(5)]
    slabs1 = [rows[kh][:, kw:kw + 44, :]
              for kh in range(5) for kw in range(5)]

    def c1_body(co, carry):
        acc = w1_ref[co, 0] * slabs1[0]
        for f in range(1, 25):
            acc = acc + w1_ref[co, f] * slabs1[f]
        p = _pool2x2(jnp.maximum(acc + b1_ref[co, 0], 0.0))
        a1_ref[pl.ds(co, 1)] = p[None]
        return carry

    jax.lax.fori_loop(0, 6, c1_body, 0)

    def c2_body(co, carry):
        acc = None
        for kh in range(3):
            for kw in range(3):
                for ci in range(6):
                    f = (kh * 3 + kw) * 6 + ci
                    t = w2_ref[co, f] * a1_ref[ci, kh:kh + 20, kw:kw + 20, :]
                    acc = t if acc is None else acc + t
        p = _pool2x2(jnp.maximum(acc + b2_ref[co, 0], 0.0))
        a2_ref[pl.ds(co, 1)] = p[None]
        return carry

    jax.lax.fori_loop(0, 6, c2_body, 0)

    def c3_body(co, carry):
        acc = None
        for kh in range(3):
            for kw in range(3):
                for ci in range(6):
                    f = (kh * 3 + kw) * 6 + ci
                    t = w3_ref[co, f] * a2_ref[ci, kh:kh + 8, kw:kw + 8, :]
                    acc = t if acc is None else acc + t
        p = _pool2x2(jnp.maximum(acc + b3_ref[co, 0], 0.0))
        a3_ref[pl.ds(co, 1)] = p[None]
        return carry

    jax.lax.fori_loop(0, 16, c3_body, 0)

    xf = a3_ref[...].reshape(256, B)
    h = jax.lax.dot_general(f1w_ref[...], xf, (((0,), (0,)), ((), ())),
                            preferred_element_type=jnp.float32)
    h = jnp.maximum(h + f1b_ref[...], 0.0)
    h = jax.lax.dot_general(f2w_ref[...], h, (((0,), (0,)), ((), ())),
                            preferred_element_type=jnp.float32)
    h = jnp.maximum(h + f2b_ref[...], 0.0)
    o = jax.lax.dot_general(f3w_ref[...], h, (((0,), (0,)), ((), ())),
                            preferred_element_type=jnp.float32)
    o_ref[...] = (o + f3b_ref[...]).astype(o_ref.dtype)


def _fernet_call(xt, c1w, c1b, c2w, c2b, c3w, c3b,
                 f1w, f1bc, f2w, f2bc, f3w, f3bc, *, interpret=False):
    N = xt.shape[-1]
    B = 128

    def smem(arr):
        return pl.BlockSpec(memory_space=pltpu.SMEM)

    def resident(arr):
        return pl.BlockSpec(arr.shape, lambda j: (0,) * arr.ndim)

    return pl.pallas_call(
        _fernet_kernel,
        out_shape=jax.ShapeDtypeStruct((3, N), jnp.float32),
        grid=(N // B,),
        in_specs=[pl.BlockSpec((48, 48, B), lambda j: (0, 0, j)),
                  smem(c1w), smem(c1b),
                  smem(c2w), smem(c2b),
                  smem(c3w), smem(c3b),
                  resident(f1w), resident(f1bc),
                  resident(f2w), resident(f2bc),
                  resident(f3w), resident(f3bc)],
        out_specs=pl.BlockSpec((3, B), lambda j: (0, j)),
        scratch_shapes=[pltpu.VMEM((6, 22, 22, B), jnp.float32),
                        pltpu.VMEM((6, 10, 10, B), jnp.float32),
                        pltpu.VMEM((16, 4, 4, B), jnp.float32)],
        compiler_params=pltpu.CompilerParams(
            dimension_semantics=("parallel",)),
        interpret=interpret,
    )(xt, c1w, c1b, c2w, c2b, c3w, c3b, f1w, f1bc, f2w, f2bc, f3w, f3bc)


def kernel(x, c1w, c1b, c2w, c2b, c3w, c3b, f1w, f1b, f2w, f2b, f3w, f3b):
    N = x.shape[0]
    xt = jnp.transpose(x.reshape(N, 48, 48), (1, 2, 0))
    out = _fernet_call(xt, c1w, c1b, c2w, c2b, c3w, c3b,
                       f1w, f1b.T, f2w, f2b.T, f3w, f3b.T)
    return out.T


# speedup vs baseline: 1.2036x; 1.2036x over previous
import jax
import jax.numpy as jnp
from jax.experimental import pallas as pl
from jax.experimental.pallas import tpu as pltpu


def _pool2x2(r):
    H, W, B = r.shape
    rr = r.reshape(H // 2, 2, W, B)
    a = jnp.maximum(rr[:, 0], rr[:, 1])
    aa = a.reshape(H // 2, W // 2, 2, B)
    return jnp.maximum(aa[:, :, 0, :], aa[:, :, 1, :])


def _fernet_kernel(x_ref, w1_ref, b1_ref, w2_ref, b2_ref, w3_ref, b3_ref,
                   f1w_ref, f1b_ref, f2w_ref, f2b_ref, f3w_ref, f3b_ref,
                   o_ref, s1_ref, s2_ref, s3_ref, a1_ref, a2_ref, a3_ref):
    B = x_ref.shape[-1]
    xb = x_ref[...]

    for kh in range(5):
        row = xb[kh:kh + 44, :, :]
        for kw in range(5):
            s1_ref[kh * 5 + kw] = row[:, kw:kw + 44, :]

    def c1_body(co, carry):
        acc = w1_ref[co, 0] * s1_ref[0]
        for f in range(1, 25):
            acc = acc + w1_ref[co, f] * s1_ref[f]
        p = _pool2x2(jnp.maximum(acc + b1_ref[co, 0], 0.0))
        a1_ref[pl.ds(co, 1)] = p[None]
        return carry

    jax.lax.fori_loop(0, 6, c1_body, 0)

    for ci in range(6):
        plane = a1_ref[ci]
        for kh in range(3):
            for kw in range(3):
                f = (kh * 3 + kw) * 6 + ci
                s2_ref[f] = plane[kh:kh + 20, kw:kw + 20, :]

    def c2_body(co, carry):
        acc = w2_ref[co, 0] * s2_ref[0]
        for f in range(1, 54):
            acc = acc + w2_ref[co, f] * s2_ref[f]
        p = _pool2x2(jnp.maximum(acc + b2_ref[co, 0], 0.0))
        a2_ref[pl.ds(co, 1)] = p[None]
        return carry

    jax.lax.fori_loop(0, 6, c2_body, 0)

    for ci in range(6):
        plane = a2_ref[ci]
        for kh in range(3):
            for kw in range(3):
                f = (kh * 3 + kw) * 6 + ci
                s3_ref[f] = plane[kh:kh + 8, kw:kw + 8, :]

    def c3_body(co, carry):
        acc = w3_ref[co, 0] * s3_ref[0]
        for f in range(1, 54):
            acc = acc + w3_ref[co, f] * s3_ref[f]
        p = _pool2x2(jnp.maximum(acc + b3_ref[co, 0], 0.0))
        a3_ref[pl.ds(co, 1)] = p[None]
        return carry

    jax.lax.fori_loop(0, 16, c3_body, 0)

    xf = a3_ref[...].reshape(256, B)
    h = jax.lax.dot_general(f1w_ref[...], xf, (((0,), (0,)), ((), ())),
                            preferred_element_type=jnp.float32)
    h = jnp.maximum(h + f1b_ref[...], 0.0)
    h = jax.lax.dot_general(f2w_ref[...], h, (((0,), (0,)), ((), ())),
                            preferred_element_type=jnp.float32)
    h = jnp.maximum(h + f2b_ref[...], 0.0)
    o = jax.lax.dot_general(f3w_ref[...], h, (((0,), (0,)), ((), ())),
                            preferred_element_type=jnp.float32)
    o_ref[...] = (o + f3b_ref[...]).astype(o_ref.dtype)


def _fernet_call(xt, c1w, c1b, c2w, c2b, c3w, c3b,
                 f1w, f1bc, f2w, f2bc, f3w, f3bc, *, interpret=False):
    N = xt.shape[-1]
    B = 128

    def smem(arr):
        return pl.BlockSpec(memory_space=pltpu.SMEM)

    def resident(arr):
        return pl.BlockSpec(arr.shape, lambda j: (0,) * arr.ndim)

    return pl.pallas_call(
        _fernet_kernel,
        out_shape=jax.ShapeDtypeStruct((3, N), jnp.float32),
        grid=(N // B,),
        in_specs=[pl.BlockSpec((48, 48, B), lambda j: (0, 0, j)),
                  smem(c1w), smem(c1b),
                  smem(c2w), smem(c2b),
                  smem(c3w), smem(c3b),
                  resident(f1w), resident(f1bc),
                  resident(f2w), resident(f2bc),
                  resident(f3w), resident(f3bc)],
        out_specs=pl.BlockSpec((3, B), lambda j: (0, j)),
        scratch_shapes=[pltpu.VMEM((25, 44, 44, B), jnp.float32),
                        pltpu.VMEM((54, 20, 20, B), jnp.float32),
                        pltpu.VMEM((54, 8, 8, B), jnp.float32),
                        pltpu.VMEM((6, 22, 22, B), jnp.float32),
                        pltpu.VMEM((6, 10, 10, B), jnp.float32),
                        pltpu.VMEM((16, 4, 4, B), jnp.float32)],
        compiler_params=pltpu.CompilerParams(
            dimension_semantics=("arbitrary",)),
        interpret=interpret,
    )(xt, c1w, c1b, c2w, c2b, c3w, c3b, f1w, f1bc, f2w, f2bc, f3w, f3bc)


def kernel(x, c1w, c1b, c2w, c2b, c3w, c3b, f1w, f1b, f2w, f2b, f3w, f3b):
    N = x.shape[0]
    xt = jnp.transpose(x.reshape(N, 48, 48), (1, 2, 0))
    out = _fernet_call(xt, c1w, c1b, c2w, c2b, c3w, c3b,
                       f1w, f1b.T, f2w, f2b.T, f3w, f3b.T)
    return out.T
